# baseline (speedup 1.0000x reference)
import functools

import jax
import jax.numpy as jnp
from jax.experimental import pallas as pl
from jax.experimental.pallas import tpu as pltpu


def _fold(logits, tile_idx, vt, max_sc, idx_sc, enable=None):
    local_max = jnp.max(logits, axis=1, keepdims=True)
    local_idx = (jnp.argmax(logits, axis=1).astype(jnp.int32)[:, None]
                 + tile_idx * vt)
    better = local_max > max_sc[...]
    if enable is not None:
        better = jnp.logical_and(better, enable)
    idx_sc[...] = jnp.where(better, local_idx, idx_sc[...])
    max_sc[...] = jnp.where(better, local_max, max_sc[...])


def _argmax_matmul_kernel(pos_ref, x_hbm, emb_ref, out_ref,
                          xs_sc, logits_sc, max_sc, idx_sc, sem,
                          *, vt: int, ng: int, d: int):
    i = pl.program_id(0)
    p = jax.lax.rem(i, 2)

    @pl.when(i == 0)
    def _init():
        copy = pltpu.make_async_copy(
            x_hbm.at[:, pl.ds(pos_ref[0] * d, d)], xs_sc, sem)
        copy.start()
        max_sc[...] = jnp.full_like(max_sc[...], -jnp.inf)
        idx_sc[...] = jnp.zeros_like(idx_sc[...])
        copy.wait()

    _fold(logits_sc[1 - p], i - 1, vt, max_sc, idx_sc, enable=i > 0)

    logits_sc[p] = jax.lax.dot_general(
        xs_sc[...], emb_ref[...], (((1,), (1,)), ((), ())),
        preferred_element_type=jnp.float32)

    @pl.when(i == ng - 1)
    def _done():
        _fold(logits_sc[p], i, vt, max_sc, idx_sc)
        out_ref[...] = idx_sc[...]


def kernel(embedding, x, output_pos, temperature, topp, topk, embedding_bias=None):
    v, d = embedding.shape
    b, s, _ = x.shape
    vt = 5000
    assert v % vt == 0
    ng = v // vt

    xt = x.reshape(b, s * d)
    pos = output_pos.astype(jnp.int32)

    out = pl.pallas_call(
        functools.partial(_argmax_matmul_kernel, vt=vt, ng=ng, d=d),
        grid=(ng,),
        in_specs=[
            pl.BlockSpec(memory_space=pltpu.SMEM),
            pl.BlockSpec(memory_space=pltpu.MemorySpace.HBM),
            pl.BlockSpec((vt, d), lambda i: (i, 0)),
        ],
        out_specs=pl.BlockSpec((b, 1), lambda i: (0, 0)),
        scratch_shapes=[
            pltpu.VMEM((b, d), jnp.float32),
            pltpu.VMEM((2, b, vt), jnp.float32),
            pltpu.VMEM((b, 1), jnp.float32),
            pltpu.VMEM((b, 1), jnp.int32),
            pltpu.SemaphoreType.DMA,
        ],
        out_shape=jax.ShapeDtypeStruct((b, 1), jnp.int32),
        compiler_params=pltpu.CompilerParams(
            vmem_limit_bytes=100 * 1024 * 1024),
    )(pos, xt, embedding)
    return out[:, 0]


# speedup vs baseline: 1.0237x; 1.0237x over previous
import functools

import jax
import jax.numpy as jnp
from jax.experimental import pallas as pl
from jax.experimental.pallas import tpu as pltpu


def _select_kernel(pos_ref, x_ref, out_ref):
    out_ref[...] = x_ref[...]


def _argmax_matmul_kernel(xs_ref, emb_ref, out_ref, max_sc, idx_sc,
                          *, vt: int, ng: int):
    i = pl.program_id(0)

    @pl.when(i == 0)
    def _init():
        max_sc[...] = jnp.full_like(max_sc[...], -jnp.inf)
        idx_sc[...] = jnp.zeros_like(idx_sc[...])

    logits = jax.lax.dot_general(
        xs_ref[...], emb_ref[...], (((1,), (1,)), ((), ())),
        preferred_element_type=jnp.float32)
    local_max = jnp.max(logits, axis=1, keepdims=True)
    local_idx = (jnp.argmax(logits, axis=1).astype(jnp.int32)[:, None]
                 + i * vt)
    better = local_max > max_sc[...]
    idx_sc[...] = jnp.where(better, local_idx, idx_sc[...])
    max_sc[...] = jnp.where(better, local_max, max_sc[...])

    @pl.when(i == ng - 1)
    def _done():
        out_ref[...] = idx_sc[...]


def kernel(embedding, x, output_pos, temperature, topp, topk, embedding_bias=None):
    v, d = embedding.shape
    b, s, _ = x.shape
    vt = 5000
    assert v % vt == 0
    ng = v // vt

    pos = output_pos.astype(jnp.int32)

    xs = pl.pallas_call(
        _select_kernel,
        grid_spec=pltpu.PrefetchScalarGridSpec(
            num_scalar_prefetch=1,
            grid=(1,),
            in_specs=[pl.BlockSpec((b, d),
                                   lambda i, pos_ref: (0, pos_ref[0]))],
            out_specs=pl.BlockSpec((b, d), lambda i, pos_ref: (0, 0)),
        ),
        out_shape=jax.ShapeDtypeStruct((b, d), jnp.float32),
    )(pos, x.reshape(b, s * d))

    out = pl.pallas_call(
        functools.partial(_argmax_matmul_kernel, vt=vt, ng=ng),
        grid=(ng,),
        in_specs=[
            pl.BlockSpec((b, d), lambda i: (0, 0)),
            pl.BlockSpec((vt, d), lambda i: (i, 0)),
        ],
        out_specs=pl.BlockSpec((b, 1), lambda i: (0, 0)),
        scratch_shapes=[
            pltpu.VMEM((b, 1), jnp.float32),
            pltpu.VMEM((b, 1), jnp.int32),
        ],
        out_shape=jax.ShapeDtypeStruct((b, 1), jnp.int32),
        compiler_params=pltpu.CompilerParams(
            vmem_limit_bytes=100 * 1024 * 1024),
    )(xs, embedding)
    return out[:, 0]


# speedup vs baseline: 1.0440x; 1.0199x over previous
import functools

import jax
import jax.numpy as jnp
from jax.experimental import pallas as pl
from jax.experimental.pallas import tpu as pltpu


def _argmax_matmul_kernel(pos_ref, x_ref, emb_ref, out_ref, max_sc, idx_sc,
                          *, vt: int, ng: int):
    i = pl.program_id(0)

    @pl.when(i == 0)
    def _init():
        max_sc[...] = jnp.full_like(max_sc[...], -jnp.inf)
        idx_sc[...] = jnp.zeros_like(idx_sc[...])

    logits = jax.lax.dot_general(
        x_ref[...], emb_ref[...], (((1,), (1,)), ((), ())),
        preferred_element_type=jnp.float32)
    local_max = jnp.max(logits, axis=1, keepdims=True)
    local_idx = (jnp.argmax(logits, axis=1).astype(jnp.int32)[:, None]
                 + i * vt)
    better = local_max > max_sc[...]
    idx_sc[...] = jnp.where(better, local_idx, idx_sc[...])
    max_sc[...] = jnp.where(better, local_max, max_sc[...])

    @pl.when(i == ng - 1)
    def _done():
        out_ref[...] = idx_sc[...]


def kernel(embedding, x, output_pos, temperature, topp, topk, embedding_bias=None):
    v, d = embedding.shape
    b, s, _ = x.shape
    vt = 4000
    assert v % vt == 0
    ng = v // vt

    xt = x.reshape(b, s * d)
    pos = output_pos.astype(jnp.int32)

    grid_spec = pltpu.PrefetchScalarGridSpec(
        num_scalar_prefetch=1,
        grid=(ng,),
        in_specs=[
            pl.BlockSpec((b, d), lambda i, pos_ref: (0, pos_ref[0])),
            pl.BlockSpec((vt, d), lambda i, pos_ref: (i, 0)),
        ],
        out_specs=pl.BlockSpec((b, 1), lambda i, pos_ref: (0, 0)),
        scratch_shapes=[
            pltpu.VMEM((b, 1), jnp.float32),
            pltpu.VMEM((b, 1), jnp.int32),
        ],
    )
    out = pl.pallas_call(
        functools.partial(_argmax_matmul_kernel, vt=vt, ng=ng),
        grid_spec=grid_spec,
        out_shape=jax.ShapeDtypeStruct((b, 1), jnp.int32),
        compiler_params=pltpu.CompilerParams(
            vmem_limit_bytes=100 * 1024 * 1024),
    )(pos, xt, embedding)
    return out[:, 0]


# speedup vs baseline: 1.2251x; 1.1735x over previous
import functools

import jax
import jax.numpy as jnp
from jax.experimental import pallas as pl
from jax.experimental.pallas import tpu as pltpu


def _select_kernel(pos_ref, x_ref, out_ref):
    out_ref[...] = x_ref[:, pl.ds(pos_ref[0] % 8, 1), :]


def _argmax_matmul_kernel(xs_ref, emb_ref, out_ref, max_sc, idx_sc,
                          *, vt: int, ng: int):
    i = pl.program_id(0)

    @pl.when(i == 0)
    def _init():
        max_sc[...] = jnp.full_like(max_sc[...], -jnp.inf)
        idx_sc[...] = jnp.zeros_like(idx_sc[...])

    logits = jax.lax.dot_general(
        xs_ref[:, 0, :], emb_ref[...], (((1,), (1,)), ((), ())),
        preferred_element_type=jnp.float32)
    local_max = jnp.max(logits, axis=1, keepdims=True)
    local_idx = (jnp.argmax(logits, axis=1).astype(jnp.int32)[:, None]
                 + i * vt)
    better = local_max > max_sc[...]
    idx_sc[...] = jnp.where(better, local_idx, idx_sc[...])
    max_sc[...] = jnp.where(better, local_max, max_sc[...])

    @pl.when(i == ng - 1)
    def _done():
        out_ref[...] = idx_sc[...]


def kernel(embedding, x, output_pos, temperature, topp, topk, embedding_bias=None):
    v, d = embedding.shape
    b, s, _ = x.shape
    vt = 4000
    assert v % vt == 0
    ng = v // vt

    pos = output_pos.astype(jnp.int32)

    xs = pl.pallas_call(
        _select_kernel,
        grid_spec=pltpu.PrefetchScalarGridSpec(
            num_scalar_prefetch=1,
            grid=(1,),
            in_specs=[pl.BlockSpec((b, 8, d),
                                   lambda i, pos_ref: (0, pos_ref[0] // 8, 0))],
            out_specs=pl.BlockSpec((b, 1, d), lambda i, pos_ref: (0, 0, 0)),
        ),
        out_shape=jax.ShapeDtypeStruct((b, 1, d), jnp.float32),
    )(pos, x)

    out = pl.pallas_call(
        functools.partial(_argmax_matmul_kernel, vt=vt, ng=ng),
        grid=(ng,),
        in_specs=[
            pl.BlockSpec((b, 1, d), lambda i: (0, 0, 0)),
            pl.BlockSpec((vt, d), lambda i: (i, 0)),
        ],
        out_specs=pl.BlockSpec((b, 1), lambda i: (0, 0)),
        scratch_shapes=[
            pltpu.VMEM((b, 1), jnp.float32),
            pltpu.VMEM((b, 1), jnp.int32),
        ],
        out_shape=jax.ShapeDtypeStruct((b, 1), jnp.int32),
        compiler_params=pltpu.CompilerParams(
            vmem_limit_bytes=100 * 1024 * 1024),
    )(xs, embedding)
    return out[:, 0]
